# baseline (speedup 1.0000x reference)
import functools

import jax
import jax.numpy as jnp
from jax.experimental import pallas as pl

B = 16384
S = 5
FEAT = 256
MOT = 64
H = 64
T1 = 2048
T = 256
P = 8
E = T + P


def _fts_kernel(feat_ref, wef_ref, out_ref):
    wef = wef_ref[...]
    for s in range(S):
        x = feat_ref[:, s, :]
        out_ref[:, s * H:(s + 1) * H] = jnp.dot(
            x, wef, preferred_element_type=jnp.float32)


def _main_kernel(
    ftsf_ref, ftsfp_ref, mot_ref, motp_ref, adjf_ref, adjfp_ref,
    adjm_ref, adjmp_ref,
    g5_ref, ones_r_ref, ones_c_ref,
    wem_ref, bef_ref, bem_ref, aef_ref, aem_ref,
    wt_ref, bt_ref, wd_ref, bd_ref,
    w1_ref, b1_ref, a1_ref, w2_ref, b2_ref, a2_ref, w3_ref, b3_ref, a3_ref,
    mrec_ref, frec_ref, sc0_ref, sc1_ref, sc2_ref,
):
    f32 = jnp.float32
    dot = functools.partial(jnp.dot, preferred_element_type=f32)

    ftsf = jnp.concatenate([ftsfp_ref[...], ftsf_ref[...]], axis=0)
    adjf = jnp.concatenate([adjfp_ref[...], adjf_ref[...]], axis=0)
    adjm = jnp.concatenate([adjmp_ref[...], adjm_ref[...]], axis=0)

    xm = jnp.concatenate([motp_ref[...], mot_ref[...]], axis=0)
    xm_stack = jnp.concatenate([xm[:, s, :] for s in range(S)], axis=0)
    fts_m = dot(xm_stack, wem_ref[...])

    bef = bef_ref[...]
    bem = bem_ref[...]
    aef = aef_ref[0, 0]
    aem = aem_ref[0, 0]
    g5 = g5_ref[...]
    hf = []
    hm = []
    for s in range(S):
        bcf = dot(adjf[:, s, :], g5)
        bcm = dot(adjm[:, s, :], g5)
        accf = bcf[:, :H] * ftsf[:, :H]
        accm = bcm[:, :H] * fts_m[:E]
        for t in range(1, S):
            accf += bcf[:, t * H:(t + 1) * H] * ftsf[:, t * H:(t + 1) * H]
            accm += bcm[:, t * H:(t + 1) * H] * fts_m[t * E:(t + 1) * E]
        accf += bef
        accm += bem
        hf.append(jnp.where(accf >= 0, accf, aef * accf))
        hm.append(jnp.where(accm >= 0, accm, aem * accm))

    hm4 = hm[S - 1]
    dstack = jnp.concatenate([hm[s] - hm4 for s in range(S - 1)], axis=0)
    z = dot(dstack, wt_ref[...]) + bt_ref[0, 0]
    sc = jax.nn.sigmoid(z)
    scb = dot(sc, ones_r_ref[...])
    readout = scb[:E] * hf[0]
    for s in range(1, S - 1):
        readout += scb[s * E:(s + 1) * E] * hf[s]

    u = dot(hf[S - 1][P:, :], wd_ref[...])
    rstack = jnp.concatenate(
        [readout[P:P + T], readout[P - 1:P - 1 + T], readout[P - 2:P - 2 + T]],
        axis=0)
    ustack = jnp.concatenate([u, u, u], axis=0)
    psum = dot(ustack * rstack, ones_c_ref[...]) + bd_ref[0, 0]
    sc0_ref[...] = psum[:T]
    sc1_ref[...] = psum[T:2 * T]
    sc2_ref[...] = psum[2 * T:]

    nmf = jnp.concatenate(
        [hm[0][P:], hm[1][P:], hm[2][P:],
         hf[0][P:], hf[1][P:], hf[2][P:]], axis=1)
    x = dot(nmf, w1_ref[...]) + b1_ref[...]
    x = jnp.where(x >= 0, x, a1_ref[...] * x)
    x = dot(x, w2_ref[...]) + b2_ref[...]
    x = jnp.where(x >= 0, x, a2_ref[...] * x)
    x = dot(x, w3_ref[...]) + b3_ref[...]
    x = jnp.where(x >= 0, x, a3_ref[...] * x)
    frec_ref[...] = x[:, :FEAT]
    mrec_ref[...] = x[:, FEAT:]


def kernel(features, motifs, adj_feat, adj_motif, W_em, b_em, a_em, W_ef,
           b_ef, a_ef, Wm1, bm1, am1, Wm2, bm2, am2, Wm3, bm3, am3, Wf1, bf1,
           af1, Wf2, bf2, af2, Wf3, bf3, af3, Wt, bt, Wd, bd):
    f32 = jnp.float32

    def row(v, n):
        return v.reshape(1, n).astype(f32)

    g5 = jnp.kron(jnp.eye(S, dtype=f32), jnp.ones((1, H), f32))
    ones_r = jnp.ones((1, H), f32)
    ones_c = jnp.ones((H, 1), f32)

    zz = jnp.zeros((3 * H, H), f32)
    w1b = jnp.concatenate(
        [jnp.concatenate([Wm1.T, zz], axis=0),
         jnp.concatenate([zz, Wf1.T], axis=0)], axis=1)
    b1b = jnp.concatenate([row(bm1, H), row(bf1, H)], axis=1)
    a1b = jnp.concatenate(
        [jnp.broadcast_to(row(am1, 1), (1, H)),
         jnp.broadcast_to(row(af1, 1), (1, H))], axis=1)
    z2 = jnp.zeros((H, H), f32)
    w2b = jnp.concatenate(
        [jnp.concatenate([Wm2.T, z2], axis=0),
         jnp.concatenate([z2, Wf2.T], axis=0)], axis=1)
    b2b = jnp.concatenate([row(bm2, H), row(bf2, H)], axis=1)
    a2b = jnp.concatenate(
        [jnp.broadcast_to(row(am2, 1), (1, H)),
         jnp.broadcast_to(row(af2, 1), (1, H))], axis=1)
    w3b = jnp.concatenate(
        [jnp.concatenate([jnp.zeros((H, FEAT), f32), Wm3.T], axis=1),
         jnp.concatenate([Wf3.T, jnp.zeros((H, MOT), f32)], axis=1)],
        axis=0)
    b3b = jnp.concatenate([row(bf3, FEAT), row(bm3, MOT)], axis=1)
    a3b = jnp.concatenate(
        [jnp.broadcast_to(row(af3, 1), (1, FEAT)),
         jnp.broadcast_to(row(am3, 1), (1, MOT))], axis=1)

    ftsf = pl.pallas_call(
        _fts_kernel,
        grid=(B // T1,),
        in_specs=[
            pl.BlockSpec((T1, S, FEAT), lambda t: (t, 0, 0)),
            pl.BlockSpec((FEAT, H), lambda t: (0, 0)),
        ],
        out_specs=pl.BlockSpec((T1, S * H), lambda t: (t, 0)),
        out_shape=jax.ShapeDtypeStruct((B, S * H), f32),
    )(features, W_ef.T)

    w_args = (
        g5, ones_r, ones_c,
        W_em.T, row(b_ef, H), row(b_em, H), row(a_ef, 1), row(a_em, 1),
        Wt.T, row(bt, 1), Wd[0], row(bd, 1),
        w1b, b1b, a1b, w2b, b2b, a2b, w3b, b3b, a3b,
    )

    grid = (B // T,)
    nb_prev = B // P

    def main2(t):
        return (t, 0)

    def prev2(t):
        return ((t * (T // P) - 1) % nb_prev, 0)

    def main3(t):
        return (t, 0, 0)

    def prev3(t):
        return ((t * (T // P) - 1) % nb_prev, 0, 0)

    def const(shape):
        return pl.BlockSpec(shape, lambda t: (0,) * len(shape))

    in_specs = [
        pl.BlockSpec((T, S * H), main2),
        pl.BlockSpec((P, S * H), prev2),
        pl.BlockSpec((T, S, MOT), main3),
        pl.BlockSpec((P, S, MOT), prev3),
        pl.BlockSpec((T, S, S), main3),
        pl.BlockSpec((P, S, S), prev3),
        pl.BlockSpec((T, S, S), main3),
        pl.BlockSpec((P, S, S), prev3),
    ] + [const(w.shape) for w in w_args]

    out_specs = [
        pl.BlockSpec((T, MOT), main2),
        pl.BlockSpec((T, FEAT), main2),
        pl.BlockSpec((T, 1), main2),
        pl.BlockSpec((T, 1), main2),
        pl.BlockSpec((T, 1), main2),
    ]
    out_shape = [
        jax.ShapeDtypeStruct((B, MOT), f32),
        jax.ShapeDtypeStruct((B, FEAT), f32),
        jax.ShapeDtypeStruct((B, 1), f32),
        jax.ShapeDtypeStruct((B, 1), f32),
        jax.ShapeDtypeStruct((B, 1), f32),
    ]

    mrec, frec, sc0, sc1, sc2 = pl.pallas_call(
        _main_kernel,
        grid=grid,
        in_specs=in_specs,
        out_specs=out_specs,
        out_shape=out_shape,
    )(ftsf, ftsf, motifs, motifs, adj_feat, adj_feat, adj_motif, adj_motif,
      *w_args)

    logits = jnp.concatenate([sc0, sc1, sc2], axis=0)
    return (logits, mrec, frec)


# speedup vs baseline: 1.3580x; 1.3580x over previous
import functools

import jax
import jax.numpy as jnp
from jax.experimental import pallas as pl

B = 16384
S = 5
FEAT = 256
MOT = 64
H = 64
T1 = 2048
T = 512
P = 8
E = T + P


def _fts_kernel(feat_ref, wef_ref, out_ref):
    wef = wef_ref[...]
    for s in range(S):
        x = feat_ref[:, s, :]
        out_ref[:, s * H:(s + 1) * H] = jnp.dot(
            x, wef, preferred_element_type=jnp.float32)


def _main_kernel(
    ftsf_ref, ftsfp_ref, mot_ref, motp_ref, adjf_ref, adjfp_ref,
    adjm_ref, adjmp_ref,
    g5_ref, ones_r_ref, ones_c_ref,
    wem_ref, bef_ref, bem_ref, aef_ref, aem_ref,
    wt_ref, bt_ref, wd_ref, bd_ref,
    w1_ref, b1_ref, a1_ref, w2_ref, b2_ref, a2_ref, w3_ref, b3_ref, a3_ref,
    mrec_ref, frec_ref, sc0_ref, sc1_ref, sc2_ref,
):
    f32 = jnp.float32
    dot = functools.partial(jnp.dot, preferred_element_type=f32)

    ftsf = jnp.concatenate([ftsfp_ref[...], ftsf_ref[...]], axis=0)
    adjf = jnp.concatenate([adjfp_ref[...], adjf_ref[...]], axis=0)
    adjm = jnp.concatenate([adjmp_ref[...], adjm_ref[...]], axis=0)

    xm = jnp.concatenate([motp_ref[...], mot_ref[...]], axis=0)
    xm_stack = jnp.concatenate(
        [xm[:, s * MOT:(s + 1) * MOT] for s in range(S)], axis=0)
    fts_m = dot(xm_stack, wem_ref[...])

    bef = bef_ref[...]
    bem = bem_ref[...]
    aef = aef_ref[0, 0]
    aem = aem_ref[0, 0]
    g5 = g5_ref[...]
    hf = []
    hm = []
    for s in range(S):
        bcf = dot(adjf[:, S * s:S * s + S], g5)
        bcm = dot(adjm[:, S * s:S * s + S], g5)
        accf = bcf[:, :H] * ftsf[:, :H]
        accm = bcm[:, :H] * fts_m[:E]
        for t in range(1, S):
            accf += bcf[:, t * H:(t + 1) * H] * ftsf[:, t * H:(t + 1) * H]
            accm += bcm[:, t * H:(t + 1) * H] * fts_m[t * E:(t + 1) * E]
        accf += bef
        accm += bem
        hf.append(jnp.where(accf >= 0, accf, aef * accf))
        hm.append(jnp.where(accm >= 0, accm, aem * accm))

    hm4 = hm[S - 1]
    dstack = jnp.concatenate([hm[s] - hm4 for s in range(S - 1)], axis=0)
    z = dot(dstack, wt_ref[...]) + bt_ref[0, 0]
    sc = jax.nn.sigmoid(z)
    scb = dot(sc, ones_r_ref[...])
    readout = scb[:E] * hf[0]
    for s in range(1, S - 1):
        readout += scb[s * E:(s + 1) * E] * hf[s]

    u = dot(hf[S - 1][P:, :], wd_ref[...])
    rstack = jnp.concatenate(
        [readout[P:P + T], readout[P - 1:P - 1 + T], readout[P - 2:P - 2 + T]],
        axis=0)
    ustack = jnp.concatenate([u, u, u], axis=0)
    psum = dot(ustack * rstack, ones_c_ref[...]) + bd_ref[0, 0]
    sc0_ref[...] = psum[:T]
    sc1_ref[...] = psum[T:2 * T]
    sc2_ref[...] = psum[2 * T:]

    nmf = jnp.concatenate(
        [hm[0][P:], hm[1][P:], hm[2][P:],
         hf[0][P:], hf[1][P:], hf[2][P:]], axis=1)
    x = dot(nmf, w1_ref[...]) + b1_ref[...]
    x = jnp.where(x >= 0, x, a1_ref[...] * x)
    x = dot(x, w2_ref[...]) + b2_ref[...]
    x = jnp.where(x >= 0, x, a2_ref[...] * x)
    x = dot(x, w3_ref[...]) + b3_ref[...]
    x = jnp.where(x >= 0, x, a3_ref[...] * x)
    frec_ref[...] = x[:, :FEAT]
    mrec_ref[...] = x[:, FEAT:]


def kernel(features, motifs, adj_feat, adj_motif, W_em, b_em, a_em, W_ef,
           b_ef, a_ef, Wm1, bm1, am1, Wm2, bm2, am2, Wm3, bm3, am3, Wf1, bf1,
           af1, Wf2, bf2, af2, Wf3, bf3, af3, Wt, bt, Wd, bd):
    f32 = jnp.float32
    mot2 = motifs.reshape(B, S * MOT)
    adjf2 = adj_feat.reshape(B, S * S)
    adjm2 = adj_motif.reshape(B, S * S)

    def row(v, n):
        return v.reshape(1, n).astype(f32)

    g5 = jnp.kron(jnp.eye(S, dtype=f32), jnp.ones((1, H), f32))
    ones_r = jnp.ones((1, H), f32)
    ones_c = jnp.ones((H, 1), f32)

    zz = jnp.zeros((3 * H, H), f32)
    w1b = jnp.concatenate(
        [jnp.concatenate([Wm1.T, zz], axis=0),
         jnp.concatenate([zz, Wf1.T], axis=0)], axis=1)
    b1b = jnp.concatenate([row(bm1, H), row(bf1, H)], axis=1)
    a1b = jnp.concatenate(
        [jnp.broadcast_to(row(am1, 1), (1, H)),
         jnp.broadcast_to(row(af1, 1), (1, H))], axis=1)
    z2 = jnp.zeros((H, H), f32)
    w2b = jnp.concatenate(
        [jnp.concatenate([Wm2.T, z2], axis=0),
         jnp.concatenate([z2, Wf2.T], axis=0)], axis=1)
    b2b = jnp.concatenate([row(bm2, H), row(bf2, H)], axis=1)
    a2b = jnp.concatenate(
        [jnp.broadcast_to(row(am2, 1), (1, H)),
         jnp.broadcast_to(row(af2, 1), (1, H))], axis=1)
    w3b = jnp.concatenate(
        [jnp.concatenate([jnp.zeros((H, FEAT), f32), Wm3.T], axis=1),
         jnp.concatenate([Wf3.T, jnp.zeros((H, MOT), f32)], axis=1)],
        axis=0)
    b3b = jnp.concatenate([row(bf3, FEAT), row(bm3, MOT)], axis=1)
    a3b = jnp.concatenate(
        [jnp.broadcast_to(row(af3, 1), (1, FEAT)),
         jnp.broadcast_to(row(am3, 1), (1, MOT))], axis=1)

    ftsf = pl.pallas_call(
        _fts_kernel,
        grid=(B // T1,),
        in_specs=[
            pl.BlockSpec((T1, S, FEAT), lambda t: (t, 0, 0)),
            pl.BlockSpec((FEAT, H), lambda t: (0, 0)),
        ],
        out_specs=pl.BlockSpec((T1, S * H), lambda t: (t, 0)),
        out_shape=jax.ShapeDtypeStruct((B, S * H), f32),
    )(features, W_ef.T)

    w_args = (
        g5, ones_r, ones_c,
        W_em.T, row(b_ef, H), row(b_em, H), row(a_ef, 1), row(a_em, 1),
        Wt.T, row(bt, 1), Wd[0], row(bd, 1),
        w1b, b1b, a1b, w2b, b2b, a2b, w3b, b3b, a3b,
    )

    grid = (B // T,)
    nb_prev = B // P

    def main2(t):
        return (t, 0)

    def prev2(t):
        return ((t * (T // P) - 1) % nb_prev, 0)

    def main3(t):
        return (t, 0, 0)

    def prev3(t):
        return ((t * (T // P) - 1) % nb_prev, 0, 0)

    def const(shape):
        return pl.BlockSpec(shape, lambda t: (0,) * len(shape))

    in_specs = [
        pl.BlockSpec((T, S * H), main2),
        pl.BlockSpec((P, S * H), prev2),
        pl.BlockSpec((T, S * MOT), main2),
        pl.BlockSpec((P, S * MOT), prev2),
        pl.BlockSpec((T, S * S), main2),
        pl.BlockSpec((P, S * S), prev2),
        pl.BlockSpec((T, S * S), main2),
        pl.BlockSpec((P, S * S), prev2),
    ] + [const(w.shape) for w in w_args]

    out_specs = [
        pl.BlockSpec((T, MOT), main2),
        pl.BlockSpec((T, FEAT), main2),
        pl.BlockSpec((T, 1), main2),
        pl.BlockSpec((T, 1), main2),
        pl.BlockSpec((T, 1), main2),
    ]
    out_shape = [
        jax.ShapeDtypeStruct((B, MOT), f32),
        jax.ShapeDtypeStruct((B, FEAT), f32),
        jax.ShapeDtypeStruct((B, 1), f32),
        jax.ShapeDtypeStruct((B, 1), f32),
        jax.ShapeDtypeStruct((B, 1), f32),
    ]

    mrec, frec, sc0, sc1, sc2 = pl.pallas_call(
        _main_kernel,
        grid=grid,
        in_specs=in_specs,
        out_specs=out_specs,
        out_shape=out_shape,
    )(ftsf, ftsf, mot2, mot2, adjf2, adjf2, adjm2, adjm2, *w_args)

    logits = jnp.concatenate([sc0, sc1, sc2], axis=0)
    return (logits, mrec, frec)


# speedup vs baseline: 1.3961x; 1.0281x over previous
import functools

import jax
import jax.numpy as jnp
from jax.experimental import pallas as pl

B = 16384
S = 5
FEAT = 256
MOT = 64
H = 64
T1 = 2048
T = 1024
P = 8
E = T + P


def _fts_kernel(feat_ref, wef_ref, out_ref):
    wef = wef_ref[...]
    for s in range(S):
        x = feat_ref[:, s, :]
        out_ref[:, s * H:(s + 1) * H] = jnp.dot(
            x, wef, preferred_element_type=jnp.float32)


def _main_kernel(
    ftsf_ref, ftsfp_ref, mot_ref, motp_ref, adjf_ref, adjfp_ref,
    adjm_ref, adjmp_ref,
    g5_ref, ones_r_ref, ones_c_ref,
    wem_ref, bef_ref, bem_ref, aef_ref, aem_ref,
    wt_ref, bt_ref, wd_ref, bd_ref,
    w1_ref, b1_ref, a1_ref, w2_ref, b2_ref, a2_ref, w3_ref, b3_ref, a3_ref,
    mrec_ref, frec_ref, sc0_ref, sc1_ref, sc2_ref,
):
    f32 = jnp.float32
    dot = functools.partial(jnp.dot, preferred_element_type=f32)

    ftsf = jnp.concatenate([ftsfp_ref[...], ftsf_ref[...]], axis=0)
    adjf = jnp.concatenate([adjfp_ref[...], adjf_ref[...]], axis=0)
    adjm = jnp.concatenate([adjmp_ref[...], adjm_ref[...]], axis=0)

    xm = jnp.concatenate([motp_ref[...], mot_ref[...]], axis=0)
    xm_stack = jnp.concatenate(
        [xm[:, s * MOT:(s + 1) * MOT] for s in range(S)], axis=0)
    fts_m = dot(xm_stack, wem_ref[...])

    bef = bef_ref[...]
    bem = bem_ref[...]
    aef = aef_ref[0, 0]
    aem = aem_ref[0, 0]
    g5 = g5_ref[...]
    hf = []
    hm = []
    for s in range(S):
        bcf = dot(adjf[:, S * s:S * s + S], g5)
        bcm = dot(adjm[:, S * s:S * s + S], g5)
        accf = bcf[:, :H] * ftsf[:, :H]
        accm = bcm[:, :H] * fts_m[:E]
        for t in range(1, S):
            accf += bcf[:, t * H:(t + 1) * H] * ftsf[:, t * H:(t + 1) * H]
            accm += bcm[:, t * H:(t + 1) * H] * fts_m[t * E:(t + 1) * E]
        accf += bef
        accm += bem
        hf.append(jnp.where(accf >= 0, accf, aef * accf))
        hm.append(jnp.where(accm >= 0, accm, aem * accm))

    hm4 = hm[S - 1]
    dstack = jnp.concatenate([hm[s] - hm4 for s in range(S - 1)], axis=0)
    z = dot(dstack, wt_ref[...]) + bt_ref[0, 0]
    sc = jax.nn.sigmoid(z)
    scb = dot(sc, ones_r_ref[...])
    readout = scb[:E] * hf[0]
    for s in range(1, S - 1):
        readout += scb[s * E:(s + 1) * E] * hf[s]

    u = dot(hf[S - 1][P:, :], wd_ref[...])
    rstack = jnp.concatenate(
        [readout[P:P + T], readout[P - 1:P - 1 + T], readout[P - 2:P - 2 + T]],
        axis=0)
    ustack = jnp.concatenate([u, u, u], axis=0)
    psum = dot(ustack * rstack, ones_c_ref[...]) + bd_ref[0, 0]
    sc0_ref[...] = psum[:T]
    sc1_ref[...] = psum[T:2 * T]
    sc2_ref[...] = psum[2 * T:]

    nmf = jnp.concatenate(
        [hm[0][P:], hm[1][P:], hm[2][P:],
         hf[0][P:], hf[1][P:], hf[2][P:]], axis=1)
    x = dot(nmf, w1_ref[...]) + b1_ref[...]
    x = jnp.where(x >= 0, x, a1_ref[...] * x)
    x = dot(x, w2_ref[...]) + b2_ref[...]
    x = jnp.where(x >= 0, x, a2_ref[...] * x)
    x = dot(x, w3_ref[...]) + b3_ref[...]
    x = jnp.where(x >= 0, x, a3_ref[...] * x)
    frec_ref[...] = x[:, :FEAT]
    mrec_ref[...] = x[:, FEAT:]


def kernel(features, motifs, adj_feat, adj_motif, W_em, b_em, a_em, W_ef,
           b_ef, a_ef, Wm1, bm1, am1, Wm2, bm2, am2, Wm3, bm3, am3, Wf1, bf1,
           af1, Wf2, bf2, af2, Wf3, bf3, af3, Wt, bt, Wd, bd):
    f32 = jnp.float32
    mot2 = motifs.reshape(B, S * MOT)
    adjf2 = adj_feat.reshape(B, S * S)
    adjm2 = adj_motif.reshape(B, S * S)

    def row(v, n):
        return v.reshape(1, n).astype(f32)

    g5 = jnp.kron(jnp.eye(S, dtype=f32), jnp.ones((1, H), f32))
    ones_r = jnp.ones((1, H), f32)
    ones_c = jnp.ones((H, 1), f32)

    zz = jnp.zeros((3 * H, H), f32)
    w1b = jnp.concatenate(
        [jnp.concatenate([Wm1.T, zz], axis=0),
         jnp.concatenate([zz, Wf1.T], axis=0)], axis=1)
    b1b = jnp.concatenate([row(bm1, H), row(bf1, H)], axis=1)
    a1b = jnp.concatenate(
        [jnp.broadcast_to(row(am1, 1), (1, H)),
         jnp.broadcast_to(row(af1, 1), (1, H))], axis=1)
    z2 = jnp.zeros((H, H), f32)
    w2b = jnp.concatenate(
        [jnp.concatenate([Wm2.T, z2], axis=0),
         jnp.concatenate([z2, Wf2.T], axis=0)], axis=1)
    b2b = jnp.concatenate([row(bm2, H), row(bf2, H)], axis=1)
    a2b = jnp.concatenate(
        [jnp.broadcast_to(row(am2, 1), (1, H)),
         jnp.broadcast_to(row(af2, 1), (1, H))], axis=1)
    w3b = jnp.concatenate(
        [jnp.concatenate([jnp.zeros((H, FEAT), f32), Wm3.T], axis=1),
         jnp.concatenate([Wf3.T, jnp.zeros((H, MOT), f32)], axis=1)],
        axis=0)
    b3b = jnp.concatenate([row(bf3, FEAT), row(bm3, MOT)], axis=1)
    a3b = jnp.concatenate(
        [jnp.broadcast_to(row(af3, 1), (1, FEAT)),
         jnp.broadcast_to(row(am3, 1), (1, MOT))], axis=1)

    ftsf = pl.pallas_call(
        _fts_kernel,
        grid=(B // T1,),
        in_specs=[
            pl.BlockSpec((T1, S, FEAT), lambda t: (t, 0, 0)),
            pl.BlockSpec((FEAT, H), lambda t: (0, 0)),
        ],
        out_specs=pl.BlockSpec((T1, S * H), lambda t: (t, 0)),
        out_shape=jax.ShapeDtypeStruct((B, S * H), f32),
    )(features, W_ef.T)

    w_args = (
        g5, ones_r, ones_c,
        W_em.T, row(b_ef, H), row(b_em, H), row(a_ef, 1), row(a_em, 1),
        Wt.T, row(bt, 1), Wd[0], row(bd, 1),
        w1b, b1b, a1b, w2b, b2b, a2b, w3b, b3b, a3b,
    )

    grid = (B // T,)
    nb_prev = B // P

    def main2(t):
        return (t, 0)

    def prev2(t):
        return ((t * (T // P) - 1) % nb_prev, 0)

    def main3(t):
        return (t, 0, 0)

    def prev3(t):
        return ((t * (T // P) - 1) % nb_prev, 0, 0)

    def const(shape):
        return pl.BlockSpec(shape, lambda t: (0,) * len(shape))

    in_specs = [
        pl.BlockSpec((T, S * H), main2),
        pl.BlockSpec((P, S * H), prev2),
        pl.BlockSpec((T, S * MOT), main2),
        pl.BlockSpec((P, S * MOT), prev2),
        pl.BlockSpec((T, S * S), main2),
        pl.BlockSpec((P, S * S), prev2),
        pl.BlockSpec((T, S * S), main2),
        pl.BlockSpec((P, S * S), prev2),
    ] + [const(w.shape) for w in w_args]

    out_specs = [
        pl.BlockSpec((T, MOT), main2),
        pl.BlockSpec((T, FEAT), main2),
        pl.BlockSpec((T, 1), main2),
        pl.BlockSpec((T, 1), main2),
        pl.BlockSpec((T, 1), main2),
    ]
    out_shape = [
        jax.ShapeDtypeStruct((B, MOT), f32),
        jax.ShapeDtypeStruct((B, FEAT), f32),
        jax.ShapeDtypeStruct((B, 1), f32),
        jax.ShapeDtypeStruct((B, 1), f32),
        jax.ShapeDtypeStruct((B, 1), f32),
    ]

    mrec, frec, sc0, sc1, sc2 = pl.pallas_call(
        _main_kernel,
        grid=grid,
        in_specs=in_specs,
        out_specs=out_specs,
        out_shape=out_shape,
    )(ftsf, ftsf, mot2, mot2, adjf2, adjf2, adjm2, adjm2, *w_args)

    logits = jnp.concatenate([sc0, sc1, sc2], axis=0)
    return (logits, mrec, frec)


# speedup vs baseline: 1.4037x; 1.0055x over previous
import functools

import jax
import jax.numpy as jnp
from jax.experimental import pallas as pl

B = 16384
S = 5
FEAT = 256
MOT = 64
H = 64
T1 = 2048
T = 2048
P = 8
E = T + P


def _fts_kernel(feat_ref, wef_ref, out_ref):
    wef = wef_ref[...]
    for s in range(S):
        x = feat_ref[:, s, :]
        out_ref[:, s * H:(s + 1) * H] = jnp.dot(
            x, wef, preferred_element_type=jnp.float32)


def _main_kernel(
    ftsf_ref, ftsfp_ref, mot_ref, motp_ref, adjf_ref, adjfp_ref,
    adjm_ref, adjmp_ref,
    g5_ref, ones_r_ref, ones_c_ref,
    wem_ref, bef_ref, bem_ref, aef_ref, aem_ref,
    wt_ref, bt_ref, wd_ref, bd_ref,
    w1_ref, b1_ref, a1_ref, w2_ref, b2_ref, a2_ref, w3_ref, b3_ref, a3_ref,
    mrec_ref, frec_ref, sc0_ref, sc1_ref, sc2_ref,
):
    f32 = jnp.float32
    dot = functools.partial(jnp.dot, preferred_element_type=f32)

    ftsf = jnp.concatenate([ftsfp_ref[...], ftsf_ref[...]], axis=0)
    adjf = jnp.concatenate([adjfp_ref[...], adjf_ref[...]], axis=0)
    adjm = jnp.concatenate([adjmp_ref[...], adjm_ref[...]], axis=0)

    xm = jnp.concatenate([motp_ref[...], mot_ref[...]], axis=0)
    xm_stack = jnp.concatenate(
        [xm[:, s * MOT:(s + 1) * MOT] for s in range(S)], axis=0)
    fts_m = dot(xm_stack, wem_ref[...])

    bef = bef_ref[...]
    bem = bem_ref[...]
    aef = aef_ref[0, 0]
    aem = aem_ref[0, 0]
    g5 = g5_ref[...]
    hf = []
    hm = []
    for s in range(S):
        bcf = dot(adjf[:, S * s:S * s + S], g5)
        bcm = dot(adjm[:, S * s:S * s + S], g5)
        accf = bcf[:, :H] * ftsf[:, :H]
        accm = bcm[:, :H] * fts_m[:E]
        for t in range(1, S):
            accf += bcf[:, t * H:(t + 1) * H] * ftsf[:, t * H:(t + 1) * H]
            accm += bcm[:, t * H:(t + 1) * H] * fts_m[t * E:(t + 1) * E]
        accf += bef
        accm += bem
        hf.append(jnp.where(accf >= 0, accf, aef * accf))
        hm.append(jnp.where(accm >= 0, accm, aem * accm))

    hm4 = hm[S - 1]
    dstack = jnp.concatenate([hm[s] - hm4 for s in range(S - 1)], axis=0)
    z = dot(dstack, wt_ref[...]) + bt_ref[0, 0]
    sc = jax.nn.sigmoid(z)
    scb = dot(sc, ones_r_ref[...])
    readout = scb[:E] * hf[0]
    for s in range(1, S - 1):
        readout += scb[s * E:(s + 1) * E] * hf[s]

    u = dot(hf[S - 1][P:, :], wd_ref[...])
    rstack = jnp.concatenate(
        [readout[P:P + T], readout[P - 1:P - 1 + T], readout[P - 2:P - 2 + T]],
        axis=0)
    ustack = jnp.concatenate([u, u, u], axis=0)
    psum = dot(ustack * rstack, ones_c_ref[...]) + bd_ref[0, 0]
    sc0_ref[...] = psum[:T]
    sc1_ref[...] = psum[T:2 * T]
    sc2_ref[...] = psum[2 * T:]

    nmf = jnp.concatenate(
        [hm[0][P:], hm[1][P:], hm[2][P:],
         hf[0][P:], hf[1][P:], hf[2][P:]], axis=1)
    x = dot(nmf, w1_ref[...]) + b1_ref[...]
    x = jnp.where(x >= 0, x, a1_ref[...] * x)
    x = dot(x, w2_ref[...]) + b2_ref[...]
    x = jnp.where(x >= 0, x, a2_ref[...] * x)
    x = dot(x, w3_ref[...]) + b3_ref[...]
    x = jnp.where(x >= 0, x, a3_ref[...] * x)
    frec_ref[...] = x[:, :FEAT]
    mrec_ref[...] = x[:, FEAT:]


def kernel(features, motifs, adj_feat, adj_motif, W_em, b_em, a_em, W_ef,
           b_ef, a_ef, Wm1, bm1, am1, Wm2, bm2, am2, Wm3, bm3, am3, Wf1, bf1,
           af1, Wf2, bf2, af2, Wf3, bf3, af3, Wt, bt, Wd, bd):
    f32 = jnp.float32
    mot2 = motifs.reshape(B, S * MOT)
    adjf2 = adj_feat.reshape(B, S * S)
    adjm2 = adj_motif.reshape(B, S * S)

    def row(v, n):
        return v.reshape(1, n).astype(f32)

    g5 = jnp.kron(jnp.eye(S, dtype=f32), jnp.ones((1, H), f32))
    ones_r = jnp.ones((1, H), f32)
    ones_c = jnp.ones((H, 1), f32)

    zz = jnp.zeros((3 * H, H), f32)
    w1b = jnp.concatenate(
        [jnp.concatenate([Wm1.T, zz], axis=0),
         jnp.concatenate([zz, Wf1.T], axis=0)], axis=1)
    b1b = jnp.concatenate([row(bm1, H), row(bf1, H)], axis=1)
    a1b = jnp.concatenate(
        [jnp.broadcast_to(row(am1, 1), (1, H)),
         jnp.broadcast_to(row(af1, 1), (1, H))], axis=1)
    z2 = jnp.zeros((H, H), f32)
    w2b = jnp.concatenate(
        [jnp.concatenate([Wm2.T, z2], axis=0),
         jnp.concatenate([z2, Wf2.T], axis=0)], axis=1)
    b2b = jnp.concatenate([row(bm2, H), row(bf2, H)], axis=1)
    a2b = jnp.concatenate(
        [jnp.broadcast_to(row(am2, 1), (1, H)),
         jnp.broadcast_to(row(af2, 1), (1, H))], axis=1)
    w3b = jnp.concatenate(
        [jnp.concatenate([jnp.zeros((H, FEAT), f32), Wm3.T], axis=1),
         jnp.concatenate([Wf3.T, jnp.zeros((H, MOT), f32)], axis=1)],
        axis=0)
    b3b = jnp.concatenate([row(bf3, FEAT), row(bm3, MOT)], axis=1)
    a3b = jnp.concatenate(
        [jnp.broadcast_to(row(af3, 1), (1, FEAT)),
         jnp.broadcast_to(row(am3, 1), (1, MOT))], axis=1)

    ftsf = pl.pallas_call(
        _fts_kernel,
        grid=(B // T1,),
        in_specs=[
            pl.BlockSpec((T1, S, FEAT), lambda t: (t, 0, 0)),
            pl.BlockSpec((FEAT, H), lambda t: (0, 0)),
        ],
        out_specs=pl.BlockSpec((T1, S * H), lambda t: (t, 0)),
        out_shape=jax.ShapeDtypeStruct((B, S * H), f32),
    )(features, W_ef.T)

    w_args = (
        g5, ones_r, ones_c,
        W_em.T, row(b_ef, H), row(b_em, H), row(a_ef, 1), row(a_em, 1),
        Wt.T, row(bt, 1), Wd[0], row(bd, 1),
        w1b, b1b, a1b, w2b, b2b, a2b, w3b, b3b, a3b,
    )

    grid = (B // T,)
    nb_prev = B // P

    def main2(t):
        return (t, 0)

    def prev2(t):
        return ((t * (T // P) - 1) % nb_prev, 0)

    def main3(t):
        return (t, 0, 0)

    def prev3(t):
        return ((t * (T // P) - 1) % nb_prev, 0, 0)

    def const(shape):
        return pl.BlockSpec(shape, lambda t: (0,) * len(shape))

    in_specs = [
        pl.BlockSpec((T, S * H), main2),
        pl.BlockSpec((P, S * H), prev2),
        pl.BlockSpec((T, S * MOT), main2),
        pl.BlockSpec((P, S * MOT), prev2),
        pl.BlockSpec((T, S * S), main2),
        pl.BlockSpec((P, S * S), prev2),
        pl.BlockSpec((T, S * S), main2),
        pl.BlockSpec((P, S * S), prev2),
    ] + [const(w.shape) for w in w_args]

    out_specs = [
        pl.BlockSpec((T, MOT), main2),
        pl.BlockSpec((T, FEAT), main2),
        pl.BlockSpec((T, 1), main2),
        pl.BlockSpec((T, 1), main2),
        pl.BlockSpec((T, 1), main2),
    ]
    out_shape = [
        jax.ShapeDtypeStruct((B, MOT), f32),
        jax.ShapeDtypeStruct((B, FEAT), f32),
        jax.ShapeDtypeStruct((B, 1), f32),
        jax.ShapeDtypeStruct((B, 1), f32),
        jax.ShapeDtypeStruct((B, 1), f32),
    ]

    mrec, frec, sc0, sc1, sc2 = pl.pallas_call(
        _main_kernel,
        grid=grid,
        in_specs=in_specs,
        out_specs=out_specs,
        out_shape=out_shape,
    )(ftsf, ftsf, mot2, mot2, adjf2, adjf2, adjm2, adjm2, *w_args)

    logits = jnp.concatenate([sc0, sc1, sc2], axis=0)
    return (logits, mrec, frec)
